# baseline (speedup 1.0000x reference)
import functools

import jax
import jax.numpy as jnp
from jax import lax
from jax.experimental import pallas as pl
from jax.experimental.pallas import tpu as pltpu
from jax.experimental.pallas import tpu_sc as plsc

N_ROWS = 320000
N_DIM = 128
N_IMG = 10000
N_IMG_PAD = 10240

NC = 2
NS = 16
NW = NC * NS

ROWS_PER_TILE = N_ROWS // NW
SCATTER_B = 80
CHUNK_B = 1
CHUNK = SCATTER_B * CHUNK_B
N_CHUNKS = ROWS_PER_TILE // CHUNK
IMG_PER_TILE = N_IMG_PAD // NS
NBUF = 4


def _sc_partial_sums(x, idx3d, zeros):
    mesh = plsc.VectorSubcoreMesh(core_axis_name="c", subcore_axis_name="s")

    @functools.partial(
        pl.kernel,
        out_type=jax.ShapeDtypeStruct((NC, N_IMG_PAD, N_DIM), jnp.float32),
        mesh=mesh,
        scratch_types=[
            pltpu.VMEM((NBUF, CHUNK, N_DIM), jnp.float32),
            pltpu.VMEM((NBUF, CHUNK_B, SCATTER_B), jnp.int32),
            pltpu.VMEM_SHARED((N_IMG_PAD, N_DIM), jnp.float32),
            pltpu.SemaphoreType.DMA((NBUF,)),
            pltpu.SemaphoreType.DMA((NBUF,)),
        ],
    )
    def body(x_hbm, idx_hbm, zeros_hbm, out_hbm, xring, iring, acc,
             seml, sems):
        c = lax.axis_index("c")
        s = lax.axis_index("s")
        wid = c * NS + s

        def start_load(g):
            b = lax.rem(g, jnp.int32(NBUF))
            grp = wid * jnp.int32(N_CHUNKS) + g
            row0 = grp * jnp.int32(CHUNK)
            pltpu.async_copy(x_hbm.at[pl.ds(row0, CHUNK)], xring.at[b],
                             seml.at[b])
            pltpu.async_copy(idx_hbm.at[pl.ds(grp, 1)], iring.at[pl.ds(b, 1)],
                             seml.at[b])

        def wait_load(g):
            b = lax.rem(g, jnp.int32(NBUF))
            grp = wid * jnp.int32(N_CHUNKS) + g
            row0 = grp * jnp.int32(CHUNK)
            pltpu.make_async_copy(x_hbm.at[pl.ds(row0, CHUNK)], xring.at[b],
                                  seml.at[b]).wait()
            pltpu.make_async_copy(idx_hbm.at[pl.ds(grp, 1)],
                                  iring.at[pl.ds(b, 1)], seml.at[b]).wait()

        def fire_scatter(g):
            b = lax.rem(g, jnp.int32(NBUF))
            pltpu.async_copy(xring.at[b], acc.at[iring.at[b, jnp.int32(0)]],
                             sems.at[b], add=True)

        def drain_scatter(g):
            b = lax.rem(g, jnp.int32(NBUF))
            pltpu.make_async_copy(xring.at[b],
                                  acc.at[iring.at[b, jnp.int32(0)]],
                                  sems.at[b]).wait()

        start_load(jnp.int32(0))
        start_load(jnp.int32(1))
        pltpu.sync_copy(zeros_hbm, acc.at[pl.ds(s * IMG_PER_TILE, IMG_PER_TILE)])
        plsc.subcore_barrier()

        def step(g, carry):
            @pl.when((g >= jnp.int32(2)) & (g < jnp.int32(N_CHUNKS - 2)))
            def _():
                drain_scatter(g - jnp.int32(2))

            @pl.when(g < jnp.int32(N_CHUNKS - 2))
            def _():
                start_load(g + jnp.int32(2))

            wait_load(g)
            fire_scatter(g)
            return carry

        lax.fori_loop(jnp.int32(0), jnp.int32(N_CHUNKS), step, jnp.int32(0))

        for g in (N_CHUNKS - 4, N_CHUNKS - 3, N_CHUNKS - 2, N_CHUNKS - 1):
            drain_scatter(jnp.int32(g))

        plsc.subcore_barrier()
        pltpu.sync_copy(
            acc.at[pl.ds(s * IMG_PER_TILE, IMG_PER_TILE)],
            out_hbm.at[c, pl.ds(s * IMG_PER_TILE, IMG_PER_TILE)],
        )

    return body(x, idx3d, zeros)


def _combine_kernel(p_ref, c_ref, o_ref):
    o_ref[...] = (p_ref[0] + p_ref[1]) / c_ref[...]


def _combine(partials, counts):
    blk = 2000
    return pl.pallas_call(
        _combine_kernel,
        out_shape=jax.ShapeDtypeStruct((N_IMG, N_DIM), jnp.float32),
        grid=(N_IMG // blk,),
        in_specs=[
            pl.BlockSpec((NC, blk, N_DIM),
                         lambda i: (jnp.int32(0), i, jnp.int32(0))),
            pl.BlockSpec((blk, 1), lambda i: (i, jnp.int32(0))),
        ],
        out_specs=pl.BlockSpec((blk, N_DIM), lambda i: (i, jnp.int32(0))),
    )(partials, counts.reshape(N_IMG, 1))


def kernel(x, image_indices, counts):
    idx3d = image_indices.astype(jnp.int32).reshape(
        N_ROWS // CHUNK, CHUNK_B, SCATTER_B)
    zeros = jnp.zeros((IMG_PER_TILE, N_DIM), jnp.float32)
    partials = _sc_partial_sums(x, idx3d, zeros)
    return _combine(partials, counts.astype(jnp.float32))


# speedup vs baseline: 1.0074x; 1.0074x over previous
import functools

import jax
import jax.numpy as jnp
from jax import lax
from jax.experimental import pallas as pl
from jax.experimental.pallas import tpu as pltpu
from jax.experimental.pallas import tpu_sc as plsc

N_ROWS = 320000
N_DIM = 128
N_IMG = 10000
N_IMG_PAD = 10240

NC = 2
NS = 16
NW = NC * NS

ROWS_PER_TILE = N_ROWS // NW
SCATTER_B = 80
CHUNK_B = 1
CHUNK = SCATTER_B * CHUNK_B
N_CHUNKS = ROWS_PER_TILE // CHUNK
IMG_PER_TILE = N_IMG_PAD // NS
NBUF = 4


def _sc_partial_sums(x, idx1d, zeros):
    mesh = plsc.VectorSubcoreMesh(core_axis_name="c", subcore_axis_name="s")

    @functools.partial(
        pl.kernel,
        out_type=jax.ShapeDtypeStruct((NC, N_IMG_PAD, N_DIM), jnp.float32),
        mesh=mesh,
        scratch_types=[
            pltpu.VMEM((NBUF, CHUNK, N_DIM), jnp.float32),
            pltpu.VMEM((NBUF, CHUNK_B, SCATTER_B), jnp.int32),
            pltpu.VMEM_SHARED((N_IMG_PAD, N_DIM), jnp.float32),
            pltpu.SemaphoreType.DMA((NBUF,)),
            pltpu.SemaphoreType.DMA((NBUF,)),
        ],
    )
    def body(x_hbm, idx_hbm, zeros_hbm, out_hbm, xring, iring, acc,
             seml, sems):
        c = lax.axis_index("c")
        s = lax.axis_index("s")
        wid = c * NS + s

        def start_load(g):
            b = lax.rem(g, jnp.int32(NBUF))
            grp = wid * jnp.int32(N_CHUNKS) + g
            row0 = grp * jnp.int32(CHUNK)
            pltpu.async_copy(x_hbm.at[pl.ds(row0, CHUNK)], xring.at[b],
                             seml.at[b])
            pltpu.async_copy(idx_hbm.at[pl.ds(row0, CHUNK)],
                             iring.at[b, jnp.int32(0)], seml.at[b])

        def wait_load(g):
            b = lax.rem(g, jnp.int32(NBUF))
            grp = wid * jnp.int32(N_CHUNKS) + g
            row0 = grp * jnp.int32(CHUNK)
            pltpu.make_async_copy(x_hbm.at[pl.ds(row0, CHUNK)], xring.at[b],
                                  seml.at[b]).wait()
            pltpu.make_async_copy(idx_hbm.at[pl.ds(row0, CHUNK)],
                                  iring.at[b, jnp.int32(0)], seml.at[b]).wait()

        def fire_scatter(g):
            b = lax.rem(g, jnp.int32(NBUF))
            pltpu.async_copy(xring.at[b], acc.at[iring.at[b, jnp.int32(0)]],
                             sems.at[b], add=True)

        def drain_scatter(g):
            b = lax.rem(g, jnp.int32(NBUF))
            pltpu.make_async_copy(xring.at[b],
                                  acc.at[iring.at[b, jnp.int32(0)]],
                                  sems.at[b]).wait()

        start_load(jnp.int32(0))
        start_load(jnp.int32(1))
        pltpu.sync_copy(zeros_hbm, acc.at[pl.ds(s * IMG_PER_TILE, IMG_PER_TILE)])
        plsc.subcore_barrier()

        def step(g, carry):
            @pl.when((g >= jnp.int32(2)) & (g < jnp.int32(N_CHUNKS - 2)))
            def _():
                drain_scatter(g - jnp.int32(2))

            @pl.when(g < jnp.int32(N_CHUNKS - 2))
            def _():
                start_load(g + jnp.int32(2))

            wait_load(g)
            fire_scatter(g)
            return carry

        lax.fori_loop(jnp.int32(0), jnp.int32(N_CHUNKS), step, jnp.int32(0))

        for g in (N_CHUNKS - 4, N_CHUNKS - 3, N_CHUNKS - 2, N_CHUNKS - 1):
            drain_scatter(jnp.int32(g))

        plsc.subcore_barrier()
        pltpu.sync_copy(
            acc.at[pl.ds(s * IMG_PER_TILE, IMG_PER_TILE)],
            out_hbm.at[c, pl.ds(s * IMG_PER_TILE, IMG_PER_TILE)],
        )

    return body(x, idx1d, zeros)


def _combine_kernel(p_ref, c_ref, o_ref):
    o_ref[...] = (p_ref[0] + p_ref[1]) / c_ref[...]


def _combine(partials, counts):
    blk = 2000
    return pl.pallas_call(
        _combine_kernel,
        out_shape=jax.ShapeDtypeStruct((N_IMG, N_DIM), jnp.float32),
        grid=(N_IMG // blk,),
        in_specs=[
            pl.BlockSpec((NC, blk, N_DIM),
                         lambda i: (jnp.int32(0), i, jnp.int32(0))),
            pl.BlockSpec((blk, 1), lambda i: (i, jnp.int32(0))),
        ],
        out_specs=pl.BlockSpec((blk, N_DIM), lambda i: (i, jnp.int32(0))),
    )(partials, counts.reshape(N_IMG, 1))


def kernel(x, image_indices, counts):
    idx1d = image_indices.astype(jnp.int32)
    zeros = jnp.zeros((IMG_PER_TILE, N_DIM), jnp.float32)
    partials = _sc_partial_sums(x, idx1d, zeros)
    return _combine(partials, counts.astype(jnp.float32))
